# baseline (speedup 1.0000x reference)
import functools

import jax
import jax.numpy as jnp
import numpy as np
from jax.experimental import pallas as pl
from jax.experimental.pallas import tpu as pltpu

_S = 8
_C = 128
_R = _S * _S * _S
_BB = 32
_INV_SQRT2 = 0.7071067811865476


def _erf(z):
    s = jnp.sign(z)
    a = jnp.abs(z)
    t = 1.0 / (1.0 + 0.3275911 * a)
    poly = t * (0.254829592 + t * (-0.284496736 + t * (1.421413741
               + t * (-1.453152027 + t * 1.061405429))))
    return s * (1.0 - poly * jnp.exp(-a * a))


def _fused_kernel(xs_ref, gs_ref, tall_ref, b0_ref, b1_ref, b2_ref,
                  fc1w_ref, fc1b_ref, fc2w_ref, fc2b_ref, pw_ref, pb_ref,
                  o_ref):
    xbs = [xs_ref[b].astype(jnp.bfloat16) for b in range(_BB)]
    xb = jnp.concatenate(xbs, axis=1)

    ams = [jnp.sum(xbs[b] * gs_ref[...], axis=0, keepdims=True)
           for b in range(_BB)]
    a = jnp.concatenate(ams, axis=0)

    t1 = jnp.dot(a, fc1w_ref[...], preferred_element_type=jnp.float32)
    t1 = t1 + fc1b_ref[...]
    t1 = 0.5 * t1 * (1.0 + _erf(t1 * _INV_SQRT2))
    t2 = jnp.dot(t1, fc2w_ref[...], preferred_element_type=jnp.float32)
    t2 = t2 + fc2b_ref[...]
    g0 = t2[:, 0:128]
    g1 = t2[:, 128:256]
    g2 = t2[:, 256:384]
    m = jnp.maximum(jnp.maximum(g0, g1), g2)
    e0 = jnp.exp(g0 - m)
    e1 = jnp.exp(g1 - m)
    e2 = jnp.exp(g2 - m)
    inv = 1.0 / (e0 + e1 + e2)
    g0 = e0 * inv
    g1 = e1 * inv
    g2 = e2 * inv
    gc0 = jnp.concatenate([g0[b:b + 1, :] for b in range(_BB)], axis=1)
    gc1 = jnp.concatenate([g1[b:b + 1, :] for b in range(_BB)], axis=1)
    gc2 = jnp.concatenate([g2[b:b + 1, :] for b in range(_BB)], axis=1)

    hall = jnp.dot(tall_ref[...], xb, preferred_element_type=jnp.float32)
    h0 = hall[0:_R]
    h1 = hall[_R:2 * _R]
    h2 = hall[2 * _R:3 * _R]

    bt0 = jnp.concatenate([b0_ref[...]] * _BB, axis=1)
    bt1 = jnp.concatenate([b1_ref[...]] * _BB, axis=1)
    bt2 = jnp.concatenate([b2_ref[...]] * _BB, axis=1)
    comb = (gc0 * (h0 + bt0) + gc1 * (h1 + bt1)
            + gc2 * (h2 + bt2))

    cs = jnp.concatenate(
        [comb[:, b * _C:(b + 1) * _C] for b in range(_BB)], axis=0)
    res = jnp.dot(cs.astype(jnp.bfloat16), pw_ref[...],
                  preferred_element_type=jnp.float32)
    res = res + pb_ref[...]
    for b in range(_BB):
        o_ref[b] = res[b * _R:(b + 1) * _R, :]


@functools.partial(jax.jit, static_argnames=())
def kernel(x, wh, bh, ww, bw, wd, bd, fc1_w, fc1_b, fc2_w, fc2_b,
           proj_w, proj_b):
    B = x.shape[0]
    S = _S
    x = x.astype(jnp.float32)
    xs = x.reshape(B, _R, _C)

    idx = np.arange(_R)
    f1 = np.zeros((_R, S), np.float32)
    f2 = np.zeros((_R, S), np.float32)
    f3 = np.zeros((_R, S), np.float32)
    f1[idx, idx // 64] = 1.0
    f2[idx, (idx // 8) % 8] = 1.0
    f3[idx, idx % 8] = 1.0
    F1 = jnp.asarray(f1)
    F2 = jnp.asarray(f2)
    F3 = jnp.asarray(f3)
    m_h = jnp.asarray((f1 @ f1.T) * (f3 @ f3.T))
    m_w = jnp.asarray((f1 @ f2.T) * (f3 @ f3.T))
    m_d = jnp.asarray((f1 @ f1.T) * (f3 @ f2.T))

    gcol = (F1 @ ww.sum(axis=0)[:, None] + F2 @ wh.sum(axis=0)[:, None]
            + F3 @ wd.sum(axis=0)[:, None]) / (S ** 3)
    gs = jnp.broadcast_to(gcol, (_R, _C))

    th = (F2 @ wh @ F2.T) * m_h
    tw = (F2 @ ww @ F1.T) * m_w
    td = (F2 @ wd @ F3.T) * m_d
    tall = jnp.concatenate([th, tw, td], axis=0).astype(jnp.bfloat16)

    def rows(bias):
        return jnp.broadcast_to(F2 @ bias[:, None], (_R, _C))
    b0 = rows(bh)
    b1 = rows(bw)
    b2 = rows(bd)

    mb = bh.mean() + bw.mean() + bd.mean()
    fc1b_adj = (fc1_b + mb * fc1_w.sum(axis=0)).reshape(1, -1)

    nC = fc2_w.shape[1] // 3
    fc2w_r = jnp.transpose(fc2_w.reshape(-1, nC, 3), (0, 2, 1)).reshape(-1, 3 * nC)
    fc2b_r = jnp.transpose(fc2_b.reshape(nC, 3), (1, 0)).reshape(1, 3 * nC)

    pw_b = proj_w.astype(jnp.bfloat16)
    pb_r = proj_b.reshape(1, -1)

    def const2(shape):
        return pl.BlockSpec(shape, lambda b: (0, 0))

    out = pl.pallas_call(
        _fused_kernel,
        grid=(B // _BB,),
        in_specs=[pl.BlockSpec((_BB, _R, _C), lambda b: (b, 0, 0)),
                  const2((_R, _C)),
                  const2((3 * _R, _R)),
                  const2((_R, _C)), const2((_R, _C)), const2((_R, _C)),
                  const2(fc1_w.shape),
                  const2((1, fc1_w.shape[1])),
                  const2(fc2w_r.shape),
                  const2((1, 3 * nC)),
                  const2(proj_w.shape),
                  const2((1, proj_w.shape[1]))],
        out_specs=pl.BlockSpec((_BB, _R, _C), lambda b: (b, 0, 0)),
        out_shape=jax.ShapeDtypeStruct((B, _R, _C), jnp.float32),
        compiler_params=pltpu.CompilerParams(
            dimension_semantics=("parallel",),
            vmem_limit_bytes=100 * 1024 * 1024),
    )(xs, gs, tall, b0, b1, b2, fc1_w, fc1b_adj, fc2w_r, fc2b_r, pw_b, pb_r)

    return out.reshape(B, S, S, S, proj_w.shape[1])


# speedup vs baseline: 1.0449x; 1.0449x over previous
import functools

import jax
import jax.numpy as jnp
import numpy as np
from jax.experimental import pallas as pl
from jax.experimental.pallas import tpu as pltpu

_S = 8
_C = 128
_R = _S * _S * _S
_BB = 16
_INV_SQRT2 = 0.7071067811865476


def _erf(z):
    s = jnp.sign(z)
    a = jnp.abs(z)
    t = 1.0 / (1.0 + 0.3275911 * a)
    poly = t * (0.254829592 + t * (-0.284496736 + t * (1.421413741
               + t * (-1.453152027 + t * 1.061405429))))
    return s * (1.0 - poly * jnp.exp(-a * a))


def _fused_kernel(xs_ref, gs_ref, wh_ref, ww_ref, wd_ref,
                  f2_ref, f1t_ref, f2t_ref, f3t_ref,
                  mh_ref, mw_ref, md_ref, b0_ref, b1_ref, b2_ref,
                  fc1w_ref, fc1b_ref, fc2w_ref, fc2b_ref, pw_ref, pb_ref,
                  o_ref, tall_ref):
    @pl.when(pl.program_id(0) == 0)
    def _build_tall():
        ah = jnp.dot(f2_ref[...], wh_ref[...], preferred_element_type=jnp.float32)
        aw = jnp.dot(f2_ref[...], ww_ref[...], preferred_element_type=jnp.float32)
        ad = jnp.dot(f2_ref[...], wd_ref[...], preferred_element_type=jnp.float32)
        th = jnp.dot(ah, f2t_ref[...], preferred_element_type=jnp.float32)
        tw = jnp.dot(aw, f1t_ref[...], preferred_element_type=jnp.float32)
        td = jnp.dot(ad, f3t_ref[...], preferred_element_type=jnp.float32)
        tall_ref[0:_R] = (th * mh_ref[...]).astype(jnp.bfloat16)
        tall_ref[_R:2 * _R] = (tw * mw_ref[...]).astype(jnp.bfloat16)
        tall_ref[2 * _R:3 * _R] = (td * md_ref[...]).astype(jnp.bfloat16)

    xs_cat = jnp.concatenate([xs_ref[b] for b in range(_BB)], axis=1)
    xb = xs_cat.astype(jnp.bfloat16)

    ams = [jnp.sum(xs_ref[b] * gs_ref[...], axis=0, keepdims=True)
           for b in range(_BB)]
    a = jnp.concatenate(ams, axis=0)

    t1 = jnp.dot(a, fc1w_ref[...], preferred_element_type=jnp.float32)
    t1 = t1 + fc1b_ref[...]
    t1 = 0.5 * t1 * (1.0 + _erf(t1 * _INV_SQRT2))
    t2 = jnp.dot(t1, fc2w_ref[...], preferred_element_type=jnp.float32)
    t2 = t2 + fc2b_ref[...]
    g0 = t2[:, 0:128]
    g1 = t2[:, 128:256]
    g2 = t2[:, 256:384]
    m = jnp.maximum(jnp.maximum(g0, g1), g2)
    e0 = jnp.exp(g0 - m)
    e1 = jnp.exp(g1 - m)
    e2 = jnp.exp(g2 - m)
    inv = 1.0 / (e0 + e1 + e2)
    g0 = e0 * inv
    g1 = e1 * inv
    g2 = e2 * inv
    gc0 = jnp.concatenate([g0[b:b + 1, :] for b in range(_BB)], axis=1)
    gc1 = jnp.concatenate([g1[b:b + 1, :] for b in range(_BB)], axis=1)
    gc2 = jnp.concatenate([g2[b:b + 1, :] for b in range(_BB)], axis=1)

    hall = jnp.dot(tall_ref[...], xb, preferred_element_type=jnp.float32)
    h0 = hall[0:_R]
    h1 = hall[_R:2 * _R]
    h2 = hall[2 * _R:3 * _R]

    bt0 = jnp.concatenate([b0_ref[...]] * _BB, axis=1)
    bt1 = jnp.concatenate([b1_ref[...]] * _BB, axis=1)
    bt2 = jnp.concatenate([b2_ref[...]] * _BB, axis=1)
    comb = (gc0 * (h0 + bt0) + gc1 * (h1 + bt1)
            + gc2 * (h2 + bt2))

    cs = jnp.concatenate(
        [comb[:, b * _C:(b + 1) * _C] for b in range(_BB)], axis=0)
    res = jnp.dot(cs.astype(jnp.bfloat16), pw_ref[...],
                  preferred_element_type=jnp.float32)
    res = res + pb_ref[...]
    for b in range(_BB):
        o_ref[b] = res[b * _R:(b + 1) * _R, :]


@functools.partial(jax.jit, static_argnames=())
def kernel(x, wh, bh, ww, bw, wd, bd, fc1_w, fc1_b, fc2_w, fc2_b,
           proj_w, proj_b):
    B = x.shape[0]
    S = _S
    x = x.astype(jnp.float32)
    xs = x.reshape(B, _R, _C)

    idx = np.arange(_R)
    f1 = np.zeros((_R, S), np.float32)
    f2 = np.zeros((_R, S), np.float32)
    f3 = np.zeros((_R, S), np.float32)
    f1[idx, idx // 64] = 1.0
    f2[idx, (idx // 8) % 8] = 1.0
    f3[idx, idx % 8] = 1.0
    F1 = jnp.asarray(f1)
    F2 = jnp.asarray(f2)
    F3 = jnp.asarray(f3)
    m_h = jnp.asarray((f1 @ f1.T) * (f3 @ f3.T))
    m_w = jnp.asarray((f1 @ f2.T) * (f3 @ f3.T))
    m_d = jnp.asarray((f1 @ f1.T) * (f3 @ f2.T))

    gcol = (F1 @ ww.sum(axis=0)[:, None] + F2 @ wh.sum(axis=0)[:, None]
            + F3 @ wd.sum(axis=0)[:, None]) / (S ** 3)
    gs = jnp.broadcast_to(gcol, (_R, _C))


    def rows(bias):
        return jnp.broadcast_to(F2 @ bias[:, None], (_R, _C))
    b0 = rows(bh)
    b1 = rows(bw)
    b2 = rows(bd)

    mb = bh.mean() + bw.mean() + bd.mean()
    fc1b_adj = (fc1_b + mb * fc1_w.sum(axis=0)).reshape(1, -1)

    nC = fc2_w.shape[1] // 3
    fc2w_r = jnp.transpose(fc2_w.reshape(-1, nC, 3), (0, 2, 1)).reshape(-1, 3 * nC)
    fc2b_r = jnp.transpose(fc2_b.reshape(nC, 3), (1, 0)).reshape(1, 3 * nC)

    pw_b = proj_w.astype(jnp.bfloat16)
    pb_r = proj_b.reshape(1, -1)

    def const2(shape):
        return pl.BlockSpec(shape, lambda b: (0, 0))

    out = pl.pallas_call(
        _fused_kernel,
        grid=(B // _BB,),
        in_specs=[pl.BlockSpec((_BB, _R, _C), lambda b: (b, 0, 0)),
                  const2((_R, _C)),
                  const2((S, S)), const2((S, S)), const2((S, S)),
                  const2((_R, S)),
                  const2((S, _R)), const2((S, _R)), const2((S, _R)),
                  const2((_R, _R)), const2((_R, _R)), const2((_R, _R)),
                  const2((_R, _C)), const2((_R, _C)), const2((_R, _C)),
                  const2(fc1_w.shape),
                  const2((1, fc1_w.shape[1])),
                  const2(fc2w_r.shape),
                  const2((1, 3 * nC)),
                  const2(proj_w.shape),
                  const2((1, proj_w.shape[1]))],
        out_specs=pl.BlockSpec((_BB, _R, _C), lambda b: (b, 0, 0)),
        out_shape=jax.ShapeDtypeStruct((B, _R, _C), jnp.float32),
        scratch_shapes=[pltpu.VMEM((3 * _R, _R), jnp.bfloat16)],
        compiler_params=pltpu.CompilerParams(
            dimension_semantics=("arbitrary",),
            vmem_limit_bytes=100 * 1024 * 1024),
    )(xs, gs, wh, ww, wd, F2, F1.T, F2.T, F3.T, m_h, m_w, m_d,
      b0, b1, b2, fc1_w, fc1b_adj, fc2w_r, fc2b_r, pw_b, pb_r)

    return out.reshape(B, S, S, S, proj_w.shape[1])
